# baseline (speedup 1.0000x reference)
import functools

import jax
import jax.numpy as jnp
from jax import lax
from jax.experimental import pallas as pl
from jax.experimental.pallas import tpu as pltpu
from jax.experimental.pallas import tpu_sc as plsc

NUM_EMB = 1_000_000
DIM = 32
BATCH = 16384

NUM_CORES = 2
NUM_SUBCORES = 16
NUM_LANES = 16
NUM_WORKERS = NUM_CORES * NUM_SUBCORES
B_PER_W = BATCH // NUM_WORKERS
SUB = 8
DIM_TILES = DIM // SUB
WIN = 128
GRP = 16
ROUND = 64
NUM_ROUNDS = B_PER_W // ROUND
CHUNKS = ROUND // NUM_LANES


@functools.partial(
    pl.kernel,
    mesh=plsc.VectorSubcoreMesh(core_axis_name="c", subcore_axis_name="s"),
    out_type=jax.ShapeDtypeStruct((DIM_TILES, SUB, BATCH), jnp.float32),
    scratch_types=[
        pltpu.VMEM((B_PER_W,), jnp.int32),
        pltpu.VMEM((DIM_TILES, SUB, ROUND * GRP), jnp.float32),
        pltpu.VMEM((DIM_TILES, SUB, ROUND * GRP), jnp.float32),
        pltpu.VMEM((DIM_TILES, SUB, B_PER_W), jnp.float32),
        pltpu.SemaphoreType.DMA,
        pltpu.SemaphoreType.DMA,
    ],
    compiler_params=pltpu.CompilerParams(needs_layout_passes=False),
)
def _gather_sc(idx_hbm, table_hbm, out_hbm, idx_v, buf0, buf1, rows_v, sem0, sem1):
    wid = lax.axis_index("s") * NUM_CORES + lax.axis_index("c")
    base = wid * B_PER_W

    pltpu.sync_copy(idx_hbm.at[pl.ds(base, B_PER_W)], idx_v)
    sems = {id(buf0): sem0, id(buf1): sem1}

    lane = lax.iota(jnp.int32, NUM_LANES)

    def fetch(g, buf):
        def row_body(i, carry2):
            chunk = idx_v[pl.ds(g * ROUND + ((i >> 4) << 4), NUM_LANES)]
            r = jnp.max(jnp.where(lane == (i & 15), chunk, 0))
            rq = pl.multiple_of((r >> 7) << 7, 128)
            m = (r >> 4) & 7

            def mk_branch(mm):
                def branch():
                    pltpu.async_copy(
                        table_hbm.at[:, :, pl.ds(rq, WIN)].at[
                            :, :, pl.ds(mm * GRP, GRP)
                        ],
                        buf.at[:, :, pl.ds(i * GRP, GRP)],
                        sems[id(buf)],
                    )

                return branch

            lax.switch(m, [mk_branch(mm) for mm in range(SUB)])
            return carry2

        lax.fori_loop(0, ROUND, row_body, 0, unroll=2)

    def drain(buf):
        pltpu.make_async_copy(
            table_hbm.at[:, :, pl.ds(0, ROUND * GRP)], buf, sems[id(buf)]
        ).wait()

    def extract(g, buf):
        def ext_body(j, carry2):
            chunk = idx_v[pl.ds(g * ROUND + j * NUM_LANES, NUM_LANES)]
            pos = (j * NUM_LANES + lane) * GRP + (chunk & (GRP - 1))
            for a in range(DIM_TILES):
                a_vec = jnp.full((NUM_LANES,), a, jnp.int32)
                for s in range(SUB):
                    s_vec = jnp.full((NUM_LANES,), s, jnp.int32)
                    vals = plsc.load_gather(buf, [a_vec, s_vec, pos])
                    rows_v[
                        a, s, pl.ds(g * ROUND + j * NUM_LANES, NUM_LANES)
                    ] = vals
            return carry2

        lax.fori_loop(0, CHUNKS, ext_body, 0)

    fetch(0, buf0)

    def pair_body(t, carry):
        g_odd = 2 * t + 1
        fetch(g_odd, buf1)
        drain(buf0)
        extract(g_odd - 1, buf0)
        g_even = 2 * t + 2
        fetch(g_even, buf0)
        drain(buf1)
        extract(g_even - 1, buf1)
        return carry

    lax.fori_loop(0, (NUM_ROUNDS - 2) // 2, pair_body, 0)
    g_last = NUM_ROUNDS - 1
    fetch(g_last, buf1)
    drain(buf0)
    extract(g_last - 1, buf0)
    drain(buf1)
    extract(g_last, buf1)

    pltpu.sync_copy(rows_v, out_hbm.at[:, :, pl.ds(base, B_PER_W)])


def kernel(indices, table):
    tbl = table.T.reshape(DIM_TILES, SUB, NUM_EMB)
    out = _gather_sc(indices.astype(jnp.int32), tbl)
    return out.reshape(DIM, BATCH).T


# speedup vs baseline: 1.1640x; 1.1640x over previous
import functools

import jax
import jax.numpy as jnp
from jax import lax
from jax.experimental import pallas as pl
from jax.experimental.pallas import tpu as pltpu
from jax.experimental.pallas import tpu_sc as plsc

NUM_EMB = 1_000_000
DIM = 32
BATCH = 16384

NUM_CORES = 2
NUM_SUBCORES = 16
NUM_LANES = 16
NUM_WORKERS = NUM_CORES * NUM_SUBCORES
B_PER_W = BATCH // NUM_WORKERS
SUB = 8
DIM_TILES = DIM // SUB
WIN = 128
GRP = 16
ROUND = 64
NUM_ROUNDS = B_PER_W // ROUND
CHUNKS = ROUND // NUM_LANES


@functools.partial(
    pl.kernel,
    mesh=plsc.VectorSubcoreMesh(core_axis_name="c", subcore_axis_name="s"),
    out_type=jax.ShapeDtypeStruct((DIM_TILES, SUB, BATCH), jnp.float32),
    scratch_types=[
        pltpu.VMEM((B_PER_W,), jnp.int32),
        pltpu.VMEM((DIM_TILES, SUB, ROUND * GRP), jnp.float32),
        pltpu.VMEM((DIM_TILES, SUB, ROUND * GRP), jnp.float32),
        pltpu.VMEM((DIM_TILES, SUB, B_PER_W), jnp.float32),
        pltpu.VMEM((SUB, ROUND + NUM_LANES), jnp.int32),
        pltpu.VMEM((SUB, ROUND + NUM_LANES), jnp.int32),
        pltpu.SemaphoreType.DMA,
        pltpu.SemaphoreType.DMA,
    ],
    compiler_params=pltpu.CompilerParams(needs_layout_passes=False),
)
def _gather_sc(
    idx_hbm, table_hbm, out_hbm, idx_v, buf0, buf1, rows_v, b_r, b_s, sem0, sem1
):
    wid = lax.axis_index("s") * NUM_CORES + lax.axis_index("c")
    base = wid * B_PER_W

    pltpu.sync_copy(idx_hbm.at[pl.ds(base, B_PER_W)], idx_v)
    sems = {id(buf0): sem0, id(buf1): sem1}

    lane = lax.iota(jnp.int32, NUM_LANES)

    def fetch(g, buf):
        def prep(j, cnts):
            chunk = idx_v[pl.ds(g * ROUND + j * NUM_LANES, NUM_LANES)]
            mv = (chunk >> 4) & 7
            slots = lane + j * NUM_LANES
            out = []
            for mm in range(SUB):
                mask = mv == mm
                plsc.store_compressed(
                    b_r.at[mm, pl.ds(cnts[mm], NUM_LANES)], chunk, mask=mask
                )
                plsc.store_compressed(
                    b_s.at[mm, pl.ds(cnts[mm], NUM_LANES)], slots, mask=mask
                )
                out.append(
                    cnts[mm] + jnp.sum(jnp.where(mask, 1, 0), dtype=jnp.int32)
                )
            return tuple(out)

        cnts = lax.fori_loop(
            0, CHUNKS, prep, tuple(jnp.int32(0) for _ in range(SUB))
        )

        for mm in range(SUB):

            def row_body(i, carry2, mm=mm):
                cbase = (i >> 4) << 4
                bchunk = b_r[mm, pl.ds(cbase, NUM_LANES)]
                schunk = b_s[mm, pl.ds(cbase, NUM_LANES)]
                sel = lane == (i & 15)
                r = jnp.max(jnp.where(sel, bchunk, 0))
                slot = jnp.max(jnp.where(sel, schunk, 0))
                rq = pl.multiple_of((r >> 7) << 7, 128)
                pltpu.async_copy(
                    table_hbm.at[:, :, pl.ds(rq, WIN)].at[
                        :, :, pl.ds(mm * GRP, GRP)
                    ],
                    buf.at[:, :, pl.ds(slot * GRP, GRP)],
                    sems[id(buf)],
                )
                return carry2

            lax.fori_loop(0, cnts[mm], row_body, 0)

    def drain(buf):
        pltpu.make_async_copy(
            table_hbm.at[:, :, pl.ds(0, ROUND * GRP)], buf, sems[id(buf)]
        ).wait()

    def extract(g, buf):
        def ext_body(j, carry2):
            chunk = idx_v[pl.ds(g * ROUND + j * NUM_LANES, NUM_LANES)]
            pos = (j * NUM_LANES + lane) * GRP + (chunk & (GRP - 1))
            for a in range(DIM_TILES):
                a_vec = jnp.full((NUM_LANES,), a, jnp.int32)
                for s in range(SUB):
                    s_vec = jnp.full((NUM_LANES,), s, jnp.int32)
                    vals = plsc.load_gather(buf, [a_vec, s_vec, pos])
                    rows_v[
                        a, s, pl.ds(g * ROUND + j * NUM_LANES, NUM_LANES)
                    ] = vals
            return carry2

        lax.fori_loop(0, CHUNKS, ext_body, 0)

    fetch(0, buf0)

    def pair_body(t, carry):
        g_odd = 2 * t + 1
        fetch(g_odd, buf1)
        drain(buf0)
        extract(g_odd - 1, buf0)
        g_even = 2 * t + 2
        fetch(g_even, buf0)
        drain(buf1)
        extract(g_even - 1, buf1)
        return carry

    lax.fori_loop(0, (NUM_ROUNDS - 2) // 2, pair_body, 0)
    g_last = NUM_ROUNDS - 1
    fetch(g_last, buf1)
    drain(buf0)
    extract(g_last - 1, buf0)
    drain(buf1)
    extract(g_last, buf1)

    pltpu.sync_copy(rows_v, out_hbm.at[:, :, pl.ds(base, B_PER_W)])


def kernel(indices, table):
    tbl = table.T.reshape(DIM_TILES, SUB, NUM_EMB)
    out = _gather_sc(indices.astype(jnp.int32), tbl)
    return out.reshape(DIM, BATCH).T


# speedup vs baseline: 1.1685x; 1.0039x over previous
import functools

import jax
import jax.numpy as jnp
from jax import lax
from jax.experimental import pallas as pl
from jax.experimental.pallas import tpu as pltpu
from jax.experimental.pallas import tpu_sc as plsc

NUM_EMB = 1_000_000
DIM = 32
BATCH = 16384

NUM_CORES = 2
NUM_SUBCORES = 16
NUM_LANES = 16
NUM_WORKERS = NUM_CORES * NUM_SUBCORES
B_PER_W = BATCH // NUM_WORKERS
SUB = 8
DIM_TILES = DIM // SUB
WIN = 128
GRP = 16
ROUND = 64
NUM_ROUNDS = B_PER_W // ROUND
CHUNKS = ROUND // NUM_LANES


@functools.partial(
    pl.kernel,
    mesh=plsc.VectorSubcoreMesh(core_axis_name="c", subcore_axis_name="s"),
    out_type=jax.ShapeDtypeStruct((DIM_TILES, SUB, BATCH), jnp.float32),
    scratch_types=[
        pltpu.VMEM((B_PER_W,), jnp.int32),
        pltpu.VMEM((DIM_TILES, SUB, ROUND * GRP), jnp.float32),
        pltpu.VMEM((DIM_TILES, SUB, ROUND * GRP), jnp.float32),
        pltpu.VMEM((DIM_TILES, SUB, B_PER_W), jnp.float32),
        pltpu.VMEM((SUB, ROUND + NUM_LANES), jnp.int32),
        pltpu.SemaphoreType.DMA,
        pltpu.SemaphoreType.DMA,
    ],
    compiler_params=pltpu.CompilerParams(needs_layout_passes=False),
)
def _gather_sc(
    idx_hbm, table_hbm, out_hbm, idx_v, buf0, buf1, rows_v, b_p, sem0, sem1
):
    wid = lax.axis_index("s") * NUM_CORES + lax.axis_index("c")
    base = wid * B_PER_W

    pltpu.sync_copy(idx_hbm.at[pl.ds(base, B_PER_W)], idx_v)
    sems = {id(buf0): sem0, id(buf1): sem1}

    lane = lax.iota(jnp.int32, NUM_LANES)

    def fetch(g, buf):
        def prep(j, cnts):
            chunk = idx_v[pl.ds(g * ROUND + j * NUM_LANES, NUM_LANES)]
            mv = (chunk >> 4) & 7
            packed = ((lane + j * NUM_LANES) << 20) | chunk
            out = []
            for mm in range(SUB):
                mask = mv == mm
                plsc.store_compressed(
                    b_p.at[mm, pl.ds(cnts[mm], NUM_LANES)], packed, mask=mask
                )
                out.append(
                    cnts[mm] + jnp.sum(jnp.where(mask, 1, 0), dtype=jnp.int32)
                )
            return tuple(out)

        cnts = lax.fori_loop(
            0, CHUNKS, prep, tuple(jnp.int32(0) for _ in range(SUB))
        )

        for mm in range(SUB):

            def row_body(i, carry2, mm=mm):
                cbase = (i >> 4) << 4
                bchunk = b_p[mm, pl.ds(cbase, NUM_LANES)]
                sel = lane == (i & 15)
                p = jnp.max(jnp.where(sel, bchunk, 0))
                r = p & jnp.int32(0xFFFFF)
                slot = p >> 20
                rq = pl.multiple_of((r >> 7) << 7, 128)
                pltpu.async_copy(
                    table_hbm.at[:, :, pl.ds(rq, WIN)].at[
                        :, :, pl.ds(mm * GRP, GRP)
                    ],
                    buf.at[:, :, pl.ds(slot * GRP, GRP)],
                    sems[id(buf)],
                )
                return carry2

            lax.fori_loop(0, cnts[mm], row_body, 0)

    def drain(buf):
        pltpu.make_async_copy(
            table_hbm.at[:, :, pl.ds(0, ROUND * GRP)], buf, sems[id(buf)]
        ).wait()

    def extract(g, buf):
        def ext_body(j, carry2):
            chunk = idx_v[pl.ds(g * ROUND + j * NUM_LANES, NUM_LANES)]
            pos = (j * NUM_LANES + lane) * GRP + (chunk & (GRP - 1))
            for a in range(DIM_TILES):
                a_vec = jnp.full((NUM_LANES,), a, jnp.int32)
                for s in range(SUB):
                    s_vec = jnp.full((NUM_LANES,), s, jnp.int32)
                    vals = plsc.load_gather(buf, [a_vec, s_vec, pos])
                    rows_v[
                        a, s, pl.ds(g * ROUND + j * NUM_LANES, NUM_LANES)
                    ] = vals
            return carry2

        lax.fori_loop(0, CHUNKS, ext_body, 0)

    fetch(0, buf0)

    def pair_body(t, carry):
        g_odd = 2 * t + 1
        fetch(g_odd, buf1)
        drain(buf0)
        extract(g_odd - 1, buf0)
        g_even = 2 * t + 2
        fetch(g_even, buf0)
        drain(buf1)
        extract(g_even - 1, buf1)
        return carry

    lax.fori_loop(0, (NUM_ROUNDS - 2) // 2, pair_body, 0)
    g_last = NUM_ROUNDS - 1
    fetch(g_last, buf1)
    drain(buf0)
    extract(g_last - 1, buf0)
    drain(buf1)
    extract(g_last, buf1)

    pltpu.sync_copy(rows_v, out_hbm.at[:, :, pl.ds(base, B_PER_W)])


def kernel(indices, table):
    tbl = table.T.reshape(DIM_TILES, SUB, NUM_EMB)
    out = _gather_sc(indices.astype(jnp.int32), tbl)
    return out.reshape(DIM, BATCH).T


# speedup vs baseline: 1.2402x; 1.0614x over previous
import functools

import jax
import jax.numpy as jnp
from jax import lax
from jax.experimental import pallas as pl
from jax.experimental.pallas import tpu as pltpu
from jax.experimental.pallas import tpu_sc as plsc

NUM_EMB = 1_000_000
DIM = 32
BATCH = 16384

NUM_CORES = 2
NUM_SUBCORES = 16
NUM_LANES = 16
NUM_WORKERS = NUM_CORES * NUM_SUBCORES
B_PER_W = BATCH // NUM_WORKERS
SUB = 8
DIM_TILES = DIM // SUB
WIN = 128
GRP = 16
ROUND = 128
NUM_ROUNDS = B_PER_W // ROUND
CHUNKS = ROUND // NUM_LANES


@functools.partial(
    pl.kernel,
    mesh=plsc.VectorSubcoreMesh(core_axis_name="c", subcore_axis_name="s"),
    out_type=jax.ShapeDtypeStruct((DIM_TILES, SUB, BATCH), jnp.float32),
    scratch_types=[
        pltpu.VMEM((B_PER_W,), jnp.int32),
        pltpu.VMEM((DIM_TILES, SUB, ROUND * GRP), jnp.float32),
        pltpu.VMEM((DIM_TILES, SUB, B_PER_W), jnp.float32),
        pltpu.VMEM((SUB, ROUND + NUM_LANES), jnp.int32),
        pltpu.SemaphoreType.DMA,
    ],
    compiler_params=pltpu.CompilerParams(needs_layout_passes=False),
)
def _gather_sc(
    idx_hbm, table_hbm, out_hbm, idx_v, buf0, rows_v, b_p, sem0
):
    wid = lax.axis_index("s") * NUM_CORES + lax.axis_index("c")
    base = wid * B_PER_W

    pltpu.sync_copy(idx_hbm.at[pl.ds(base, B_PER_W)], idx_v)
    sems = {id(buf0): sem0}

    lane = lax.iota(jnp.int32, NUM_LANES)

    def fetch(g, buf):
        def prep(j, cnts):
            chunk = idx_v[pl.ds(g * ROUND + j * NUM_LANES, NUM_LANES)]
            mv = (chunk >> 4) & 7
            packed = ((lane + j * NUM_LANES) << 20) | chunk
            out = []
            for mm in range(SUB):
                mask = mv == mm
                plsc.store_compressed(
                    b_p.at[mm, pl.ds(cnts[mm], NUM_LANES)], packed, mask=mask
                )
                out.append(
                    cnts[mm] + jnp.sum(jnp.where(mask, 1, 0), dtype=jnp.int32)
                )
            return tuple(out)

        cnts = lax.fori_loop(
            0, CHUNKS, prep, tuple(jnp.int32(0) for _ in range(SUB))
        )

        for mm in range(SUB):

            def row_body(i, carry2, mm=mm):
                cbase = (i >> 4) << 4
                bchunk = b_p[mm, pl.ds(cbase, NUM_LANES)]
                sel = lane == (i & 15)
                p = jnp.max(jnp.where(sel, bchunk, 0))
                r = p & jnp.int32(0xFFFFF)
                slot = p >> 20
                rq = pl.multiple_of((r >> 7) << 7, 128)
                pltpu.async_copy(
                    table_hbm.at[:, :, pl.ds(rq, WIN)].at[
                        :, :, pl.ds(mm * GRP, GRP)
                    ],
                    buf.at[:, :, pl.ds(slot * GRP, GRP)],
                    sems[id(buf)],
                )
                return carry2

            lax.fori_loop(0, cnts[mm], row_body, 0)

    def drain(buf):
        pltpu.make_async_copy(
            table_hbm.at[:, :, pl.ds(0, ROUND * GRP)], buf, sems[id(buf)]
        ).wait()

    def extract(g, buf):
        def ext_body(j, carry2):
            chunk = idx_v[pl.ds(g * ROUND + j * NUM_LANES, NUM_LANES)]
            pos = (j * NUM_LANES + lane) * GRP + (chunk & (GRP - 1))
            for a in range(DIM_TILES):
                a_vec = jnp.full((NUM_LANES,), a, jnp.int32)
                for s in range(SUB):
                    s_vec = jnp.full((NUM_LANES,), s, jnp.int32)
                    vals = plsc.load_gather(buf, [a_vec, s_vec, pos])
                    rows_v[
                        a, s, pl.ds(g * ROUND + j * NUM_LANES, NUM_LANES)
                    ] = vals
            return carry2

        lax.fori_loop(0, CHUNKS, ext_body, 0)

    def round_loop(g, carry):
        fetch(g, buf0)
        drain(buf0)
        extract(g, buf0)
        return carry

    lax.fori_loop(0, NUM_ROUNDS, round_loop, 0)

    pltpu.sync_copy(rows_v, out_hbm.at[:, :, pl.ds(base, B_PER_W)])


def kernel(indices, table):
    tbl = table.T.reshape(DIM_TILES, SUB, NUM_EMB)
    out = _gather_sc(indices.astype(jnp.int32), tbl)
    return out.reshape(DIM, BATCH).T
